# Initial kernel scaffold; baseline (speedup 1.0000x reference)
import jax
import jax.numpy as jnp
from jax.experimental import pallas as pl


def kernel(input, embedding_weight):
    raise NotImplementedError("write your pallas kernel here")


# speedup vs baseline: 1.5180x; 1.5180x over previous
import functools

import jax
import jax.numpy as jnp
from jax import lax
from jax.experimental import pallas as pl
from jax.experimental.pallas import tpu as pltpu
from jax.experimental.pallas import tpu_sc as plsc

_NC, _NS, _L = 2, 16, 16
_NW = _NC * _NS

_D = 16
_IDX_MINOR = 128
_GPC = 8
_CHUNK = _IDX_MINOR * _GPC


def _gather_body(idx_hbm, table_hbm, out_hbm, idxv, idx2v, rowsv, sem,
                 *, groups_per_worker):
    wid = lax.axis_index("s") * _NC + lax.axis_index("c")
    n_chunks = groups_per_worker // _GPC

    def chunk_body(c, _):
        g0 = (wid * n_chunks + c) * _GPC
        pltpu.sync_copy(idx_hbm.at[pl.ds(g0, _GPC)], idxv)

        def shift_body(t, _):
            j = t // (_IDX_MINOR // _L)
            i = (t % (_IDX_MINOR // _L)) * _L
            idx2v[j, pl.ds(i, _L)] = idxv[j, pl.ds(i, _L)] * 2
            return 0

        lax.fori_loop(0, _GPC * (_IDX_MINOR // _L), shift_body, 0)

        for j in range(_GPC):
            pltpu.async_copy(table_hbm.at[idx2v.at[j]], rowsv.at[j], sem)
        for j in range(_GPC):
            pltpu.make_async_copy(table_hbm.at[idx2v.at[j]], rowsv.at[j],
                                  sem).wait()

        pltpu.sync_copy(rowsv, out_hbm.at[pl.ds(g0, _GPC)])
        return 0

    lax.fori_loop(0, n_chunks, chunk_body, 0)


@jax.jit
def kernel(input, embedding_weight):
    B, H = input.shape
    n_emb, two_d = embedding_weight.shape
    d = two_d // 2
    n = B * H
    assert d == _D and n % (_NW * _CHUNK) == 0

    idx = input.reshape(n // _IDX_MINOR, _IDX_MINOR).astype(jnp.int32)
    table = embedding_weight.reshape(n_emb * 2, d)
    groups_per_worker = (n // _IDX_MINOR) // _NW

    mesh = plsc.VectorSubcoreMesh(core_axis_name="c", subcore_axis_name="s")
    out = pl.kernel(
        functools.partial(_gather_body, groups_per_worker=groups_per_worker),
        out_type=jax.ShapeDtypeStruct((n // _IDX_MINOR, _IDX_MINOR, d),
                                      jnp.float32),
        mesh=mesh,
        compiler_params=pltpu.CompilerParams(use_tc_tiling_on_sc=False),
        scratch_types=[
            pltpu.VMEM((_GPC, _IDX_MINOR), jnp.int32),
            pltpu.VMEM((_GPC, _IDX_MINOR), jnp.int32),
            pltpu.VMEM((_GPC, _IDX_MINOR, _D), jnp.float32),
            pltpu.SemaphoreType.DMA,
        ],
    )(idx, table)
    return out.reshape(B, H, d)
